# baseline (speedup 1.0000x reference)
import functools

import jax
import jax.numpy as jnp
from jax.experimental import pallas as pl
from jax.experimental.pallas import tpu as pltpu


def _proj_kernel(x_ref, w_ref, h_ref):
    h_ref[...] = jnp.dot(
        x_ref[...].astype(jnp.bfloat16),
        w_ref[...].astype(jnp.bfloat16),
        preferred_element_type=jnp.float32,
    )


def _agg_kernel(adj_ref, h_ref, b_ref, out_ref):
    acc = jnp.dot(adj_ref[...], h_ref[...], preferred_element_type=jnp.float32)
    out_ref[...] = acc + b_ref[...]


def kernel(input, adj, W, b):
    m, kin = input.shape
    kout = W.shape[1]
    n = adj.shape[1]

    bm_p = 2000 if m % 2000 == 0 else m
    h = pl.pallas_call(
        _proj_kernel,
        grid=(m // bm_p,),
        in_specs=[
            pl.BlockSpec((bm_p, kin), lambda i: (i, 0)),
            pl.BlockSpec((kin, kout), lambda i: (0, 0)),
        ],
        out_specs=pl.BlockSpec((bm_p, kout), lambda i: (i, 0)),
        out_shape=jax.ShapeDtypeStruct((m, kout), jnp.float32),
        compiler_params=pltpu.CompilerParams(
            dimension_semantics=("parallel",),
        ),
    )(input, W)

    bm = 200 if m % 200 == 0 else m
    b2 = b.reshape(1, kout)
    out = pl.pallas_call(
        _agg_kernel,
        grid=(m // bm,),
        in_specs=[
            pl.BlockSpec((bm, n), lambda i: (i, 0)),
            pl.BlockSpec((n, kout), lambda i: (0, 0)),
            pl.BlockSpec((1, kout), lambda i: (0, 0)),
        ],
        out_specs=pl.BlockSpec((bm, kout), lambda i: (i, 0)),
        out_shape=jax.ShapeDtypeStruct((m, kout), jnp.float32),
        compiler_params=pltpu.CompilerParams(
            dimension_semantics=("parallel",),
        ),
    )(adj, h, b2)
    return out


# speedup vs baseline: 1.1329x; 1.1329x over previous
import functools

import jax
import jax.numpy as jnp
from jax.experimental import pallas as pl
from jax.experimental.pallas import tpu as pltpu


def _proj_kernel(x_ref, w_ref, h_ref):
    h_ref[...] = jnp.dot(
        x_ref[...].astype(jnp.bfloat16),
        w_ref[...].astype(jnp.bfloat16),
        preferred_element_type=jnp.float32,
    ).astype(jnp.bfloat16)


def _agg_kernel(adj_ref, h_ref, b_ref, out_ref):
    a = adj_ref[...].astype(jnp.bfloat16)
    acc = jnp.dot(a, h_ref[...], preferred_element_type=jnp.float32)
    out_ref[...] = acc + b_ref[...]


def kernel(input, adj, W, b):
    m, kin = input.shape
    kout = W.shape[1]
    n = adj.shape[1]

    bm_p = 2000 if m % 2000 == 0 else m
    h = pl.pallas_call(
        _proj_kernel,
        grid=(m // bm_p,),
        in_specs=[
            pl.BlockSpec((bm_p, kin), lambda i: (i, 0)),
            pl.BlockSpec((kin, kout), lambda i: (0, 0)),
        ],
        out_specs=pl.BlockSpec((bm_p, kout), lambda i: (i, 0)),
        out_shape=jax.ShapeDtypeStruct((m, kout), jnp.bfloat16),
        compiler_params=pltpu.CompilerParams(
            dimension_semantics=("parallel",),
        ),
    )(input, W)

    bm = 400 if m % 400 == 0 else m
    b2 = b.reshape(1, kout)
    out = pl.pallas_call(
        _agg_kernel,
        grid=(m // bm,),
        in_specs=[
            pl.BlockSpec((bm, n), lambda i: (i, 0)),
            pl.BlockSpec((n, kout), lambda i: (0, 0)),
            pl.BlockSpec((1, kout), lambda i: (0, 0)),
        ],
        out_specs=pl.BlockSpec((bm, kout), lambda i: (i, 0)),
        out_shape=jax.ShapeDtypeStruct((m, kout), jnp.float32),
        compiler_params=pltpu.CompilerParams(
            dimension_semantics=("parallel",),
        ),
    )(adj, h, b2)
    return out


# speedup vs baseline: 1.1377x; 1.0043x over previous
import functools

import jax
import jax.numpy as jnp
from jax.experimental import pallas as pl
from jax.experimental.pallas import tpu as pltpu


def _proj_kernel(x_ref, w_ref, h_ref):
    h_ref[...] = jnp.dot(
        x_ref[...].astype(jnp.bfloat16),
        w_ref[...].astype(jnp.bfloat16),
        preferred_element_type=jnp.float32,
    ).astype(jnp.bfloat16)


def _agg_kernel(adj_ref, h_ref, b_ref, out_ref):
    a = adj_ref[...].astype(jnp.bfloat16)
    acc = jnp.dot(a, h_ref[...], preferred_element_type=jnp.float32)
    out_ref[...] = acc + b_ref[...]


def kernel(input, adj, W, b):
    m, kin = input.shape
    kout = W.shape[1]
    n = adj.shape[1]

    bm_p = 2000 if m % 2000 == 0 else m
    h = pl.pallas_call(
        _proj_kernel,
        grid=(m // bm_p,),
        in_specs=[
            pl.BlockSpec((bm_p, kin), lambda i: (i, 0)),
            pl.BlockSpec((kin, kout), lambda i: (0, 0)),
        ],
        out_specs=pl.BlockSpec((bm_p, kout), lambda i: (i, 0)),
        out_shape=jax.ShapeDtypeStruct((m, kout), jnp.bfloat16),
        compiler_params=pltpu.CompilerParams(
            dimension_semantics=("parallel",),
        ),
    )(input, W)

    bm = 512 if m > 512 else m
    b2 = b.reshape(1, kout)
    out = pl.pallas_call(
        _agg_kernel,
        grid=(pl.cdiv(m, bm),),
        in_specs=[
            pl.BlockSpec((bm, n), lambda i: (i, 0)),
            pl.BlockSpec((n, kout), lambda i: (0, 0)),
            pl.BlockSpec((1, kout), lambda i: (0, 0)),
        ],
        out_specs=pl.BlockSpec((bm, kout), lambda i: (i, 0)),
        out_shape=jax.ShapeDtypeStruct((m, kout), jnp.float32),
        compiler_params=pltpu.CompilerParams(
            dimension_semantics=("parallel",),
            vmem_limit_bytes=64 * 1024 * 1024,
        ),
    )(adj, h, b2)
    return out


# speedup vs baseline: 1.1622x; 1.0215x over previous
import functools

import jax
import jax.numpy as jnp
from jax.experimental import pallas as pl
from jax.experimental.pallas import tpu as pltpu


def _gcn_kernel(x_hbm, w_ref, adj_ref, b_ref, out_ref, h_ref, xbuf, sems,
                *, m: int, chunk: int):
    i = pl.program_id(0)

    @pl.when(i == 0)
    def _compute_h():
        w = w_ref[...].astype(jnp.bfloat16)
        nchunks = m // chunk
        cp0 = pltpu.make_async_copy(
            x_hbm.at[pl.ds(0, chunk), :], xbuf.at[0], sems.at[0])
        cp0.start()
        for c in range(nchunks):
            if c + 1 < nchunks:
                cpn = pltpu.make_async_copy(
                    x_hbm.at[pl.ds((c + 1) * chunk, chunk), :],
                    xbuf.at[(c + 1) % 2], sems.at[(c + 1) % 2])
                cpn.start()
            pltpu.make_async_copy(
                x_hbm.at[pl.ds(c * chunk, chunk), :],
                xbuf.at[c % 2], sems.at[c % 2]).wait()
            h_ref[pl.ds(c * chunk, chunk), :] = jnp.dot(
                xbuf[c % 2].astype(jnp.bfloat16), w,
                preferred_element_type=jnp.float32,
            ).astype(jnp.bfloat16)

    a = adj_ref[...].astype(jnp.bfloat16)
    acc = jnp.dot(a, h_ref[...], preferred_element_type=jnp.float32)
    out_ref[...] = acc + b_ref[...]


def kernel(input, adj, W, b):
    m, kin = input.shape
    kout = W.shape[1]
    n = adj.shape[1]

    bm = 400 if m % 400 == 0 else m
    chunk = 2000 if m % 2000 == 0 else m
    b2 = b.reshape(1, kout)

    body = functools.partial(_gcn_kernel, m=m, chunk=chunk)
    out = pl.pallas_call(
        body,
        grid=(pl.cdiv(m, bm),),
        in_specs=[
            pl.BlockSpec(memory_space=pl.ANY),
            pl.BlockSpec((kin, kout), lambda i: (0, 0)),
            pl.BlockSpec((bm, n), lambda i: (i, 0)),
            pl.BlockSpec((1, kout), lambda i: (0, 0)),
        ],
        out_specs=pl.BlockSpec((bm, kout), lambda i: (i, 0)),
        out_shape=jax.ShapeDtypeStruct((m, kout), jnp.float32),
        scratch_shapes=[
            pltpu.VMEM((n, kout), jnp.bfloat16),
            pltpu.VMEM((2, chunk, kin), jnp.float32),
            pltpu.SemaphoreType.DMA((2,)),
        ],
        compiler_params=pltpu.CompilerParams(
            dimension_semantics=("arbitrary",),
            vmem_limit_bytes=64 * 1024 * 1024,
        ),
    )(input, W, adj, b2)
    return out
